# baseline (speedup 1.0000x reference)
import functools

import jax
import jax.numpy as jnp
from jax import lax
from jax.experimental import pallas as pl
from jax.experimental.pallas import tpu as pltpu
from jax.experimental.pallas import tpu_sc as plsc

F32 = jnp.float32
I32 = jnp.int32


def _sc_gather_kernel(xt_hbm, off_hbm, z_hbm, t0_v, t1_v, t2_v, t3_v,
                      off_v, zb_v, sem_tab, sem_off, sem_z, sem_z2):
    wid = lax.axis_index("s") * 2 + lax.axis_index("c")
    b = wid // 16
    cg = (wid // 4) % 4
    hq = wid % 4
    h0 = hq * 4
    tabs = (t0_v, t1_v, t2_v, t3_v)

    for cp in range(4):
        pltpu.make_async_copy(xt_hbm.at[b * 4 + cg, cp], tabs[cp],
                              sem_tab).start()
    for cp in range(4):
        pltpu.make_async_copy(xt_hbm.at[b * 4 + cg, cp], tabs[cp],
                              sem_tab).wait()

    iota_i = lax.broadcasted_iota(I32, (16,), 0)
    iota_f = iota_i.astype(F32)

    pltpu.make_async_copy(off_hbm.at[b, h0], off_v.at[0], sem_off).start()

    def dim_stuff(p):
        t = p.astype(I32)
        fl = t - (t.astype(F32) > p).astype(I32)
        flf = fl.astype(F32)
        q0 = jnp.clip(fl, 0, 17)
        q1 = jnp.clip(fl + 1, 0, 17)
        mask = (p < 1.0) | (p > 16.0)
        pm = jnp.where(mask, flf, p)
        pm = jnp.clip(pm, 0.0, 17.0)
        g0 = 1.0 + (q0.astype(F32) - pm)
        g1 = 1.0 - (q1.astype(F32) - pm)
        return q0, q1, g0, g1

    def hp_body(hp, _):
        h = h0 + hp
        par = lax.rem(hp, 2)
        pltpu.make_async_copy(off_hbm.at[b, h], off_v.at[par], sem_off).wait()

        @pl.when(hp < 3)
        def _():
            pltpu.make_async_copy(off_hbm.at[b, h + 1],
                                  off_v.at[lax.rem(hp + 1, 2)],
                                  sem_off).start()

        hf = h.astype(F32)

        for jk in range(9):
            j, k = jk // 3, jk % 3
            pz_ = jk % 2
            sem_p = sem_z if pz_ == 0 else sem_z2

            def _reclaim():
                pltpu.make_async_copy(
                    zb_v.at[pl.ds(pz_ * 6144, 6144)],
                    z_hbm.at[b, h, jk, cg], sem_p).wait()

            if jk >= 2:
                _reclaim()
            else:
                pl.when(hp > 0)(_reclaim)

            def nj_body(nj, _):
                nrow = 9 * j + 3 * nj + k
                pyc = (nj - 1).astype(F32)

                @plsc.parallel_loop(0, 16, unroll=2)
                def g_body(g):
                    s0 = g * 16
                    offx = off_v[par, nrow, pl.ds(s0, 16)]
                    offy = off_v[par, nrow + 27, pl.ds(s0, 16)]
                    offz = off_v[par, nrow + 54, pl.ds(s0, 16)]
                    px = offx + (hf + float(j))
                    py = offy + (g.astype(F32) + pyc + 1.0)
                    pz = offz + (iota_f + float(k))
                    q0x, q1x, gx0, gx1 = dim_stuff(px)
                    q0y, q1y, gy0, gy1 = dim_stuff(py)
                    q0z, q1z, gz0, gz1 = dim_stuff(pz)
                    bx0 = q0x * 324
                    bx1 = q1x * 324
                    by0 = q0y * 18
                    by1 = q1y * 18
                    bases = []
                    wts = []
                    for bx, gx in ((bx0, gx0), (bx1, gx1)):
                        for by, gy in ((by0, gy0), (by1, gy1)):
                            bxy = bx + by
                            gxy = gx * gy
                            for bz, gz in ((q0z, gz0), (q1z, gz1)):
                                bases.append(bxy + bz)
                                wts.append(gxy * gz)
                    tidx = 3 * iota_i + (48 * g + nj) + pz_ * 6144
                    for cp in range(4):
                        w0 = plsc.load_gather(tabs[cp], [bases[0]])
                        acc0 = wts[0] * plsc.bitcast(w0 << 16, F32)
                        acc1 = wts[0] * plsc.bitcast(w0, F32)
                        for cor in range(1, 8):
                            w = plsc.load_gather(tabs[cp], [bases[cor]])
                            acc0 = acc0 + wts[cor] * plsc.bitcast(w << 16, F32)
                            acc1 = acc1 + wts[cor] * plsc.bitcast(w, F32)
                        plsc.store_scatter(zb_v, [tidx + (2 * cp) * 768], acc0)
                        plsc.store_scatter(zb_v, [tidx + (2 * cp + 1) * 768],
                                           acc1)
                return 0

            lax.fori_loop(0, 3, nj_body, 0)

            pltpu.make_async_copy(
                zb_v.at[pl.ds(pz_ * 6144, 6144)],
                z_hbm.at[b, h, jk, cg], sem_p).start()
        return 0

    lax.fori_loop(0, 4, hp_body, 0)

    for sem_p in (sem_z, sem_z2):
        pltpu.make_async_copy(zb_v.at[pl.ds(0, 6144)],
                              z_hbm.at[b, h0, 0, cg], sem_p).wait()


def _tc_matmul_body(w_ref, z_ref, o_ref):
    o_ref[0, 0] = jnp.dot(w_ref[...], z_ref[0, 0],
                          preferred_element_type=F32)


@jax.jit
def kernel(x, offset, W):
    xp = jnp.pad(x, ((0, 0), (0, 0), (1, 1), (1, 1), (1, 1)))
    xb = jax.lax.bitcast_convert_type(
        xp.astype(jnp.bfloat16), jnp.uint16).astype(jnp.uint32)
    xb = xb.reshape(2, 16, 2, 5832)
    xt = (xb[:, :, 0] | (xb[:, :, 1] << 16)).astype(jnp.int32).reshape(8, 4, 5832)
    off_t = offset.reshape(2, 81, 16, 256).transpose(0, 2, 1, 3)

    mesh = plsc.VectorSubcoreMesh(core_axis_name="c", subcore_axis_name="s")
    sc = pl.kernel(
        _sc_gather_kernel, mesh=mesh,
        compiler_params=pltpu.CompilerParams(needs_layout_passes=False),
        out_type=jax.ShapeDtypeStruct((2, 16, 9, 4, 6144), F32),
        scratch_types=[
            pltpu.VMEM((5832,), I32),
            pltpu.VMEM((5832,), I32),
            pltpu.VMEM((5832,), I32),
            pltpu.VMEM((5832,), I32),
            pltpu.VMEM((2, 81, 256), F32),
            pltpu.VMEM((12288,), F32),
            pltpu.SemaphoreType.DMA,
            pltpu.SemaphoreType.DMA,
            pltpu.SemaphoreType.DMA,
            pltpu.SemaphoreType.DMA,
        ])
    z = sc(xt, off_t)
    zr = z.reshape(2, 16, 864, 256)

    W3 = W.transpose(0, 3, 4, 1, 2).reshape(64, 864)

    out_t = pl.pallas_call(
        _tc_matmul_body,
        grid=(2, 16),
        in_specs=[
            pl.BlockSpec((64, 864), lambda b, h: (0, 0)),
            pl.BlockSpec((1, 1, 864, 256), lambda b, h: (b, h, 0, 0)),
        ],
        out_specs=pl.BlockSpec((1, 1, 64, 256), lambda b, h: (b, h, 0, 0)),
        out_shape=jax.ShapeDtypeStruct((2, 16, 64, 256), F32),
    )(W3, zr)
    return out_t.transpose(0, 2, 1, 3).reshape(2, 64, 16, 16, 16)


# speedup vs baseline: 1.1677x; 1.1677x over previous
import functools

import jax
import jax.numpy as jnp
from jax import lax
from jax.experimental import pallas as pl
from jax.experimental.pallas import tpu as pltpu
from jax.experimental.pallas import tpu_sc as plsc

F32 = jnp.float32
I32 = jnp.int32


def _sc_gather_kernel(xt_hbm, off_hbm, z_hbm, t0_v, t1_v, t2_v, t3_v,
                      off_v, zb_v, sem_tab, sem_off, sem_z, sem_z2):
    wid = lax.axis_index("s") * 2 + lax.axis_index("c")
    b = wid // 16
    cg = (wid // 4) % 4
    hq = wid % 4
    h0 = hq * 4
    tabs = (t0_v, t1_v, t2_v, t3_v)

    for cp in range(4):
        pltpu.make_async_copy(xt_hbm.at[b * 4 + cg, cp], tabs[cp],
                              sem_tab).start()
    for cp in range(4):
        pltpu.make_async_copy(xt_hbm.at[b * 4 + cg, cp], tabs[cp],
                              sem_tab).wait()

    iota_i = lax.broadcasted_iota(I32, (16,), 0)
    iota_f = iota_i.astype(F32)

    pltpu.make_async_copy(off_hbm.at[b, h0], off_v.at[0], sem_off).start()

    def dim_stuff(p):
        t = p.astype(I32)
        fl = t - (t.astype(F32) > p).astype(I32)
        flf = fl.astype(F32)
        q0 = jnp.clip(fl, 0, 17)
        q1 = jnp.clip(fl + 1, 0, 17)
        mask = (p < 1.0) | (p > 16.0)
        pm = jnp.where(mask, flf, p)
        pm = jnp.clip(pm, 0.0, 17.0)
        g0 = 1.0 + (q0.astype(F32) - pm)
        g1 = 1.0 - (q1.astype(F32) - pm)
        return q0, q1, g0, g1

    def hp_body(hp, _):
        h = h0 + hp
        par = lax.rem(hp, 2)
        pltpu.make_async_copy(off_hbm.at[b, h], off_v.at[par], sem_off).wait()

        @pl.when(hp < 3)
        def _():
            pltpu.make_async_copy(off_hbm.at[b, h + 1],
                                  off_v.at[lax.rem(hp + 1, 2)],
                                  sem_off).start()

        hf = h.astype(F32)

        for jk in range(9):
            j, k = jk // 3, jk % 3
            pz_ = jk % 2
            sem_p = sem_z if pz_ == 0 else sem_z2

            def _reclaim():
                pltpu.make_async_copy(
                    zb_v.at[pl.ds(pz_ * 6144, 6144)],
                    z_hbm.at[b, h, jk, cg], sem_p).wait()

            if jk >= 2:
                _reclaim()
            else:
                pl.when(hp > 0)(_reclaim)

            def nj_body(nj, _):
                nrow = 9 * j + 3 * nj + k
                pyc = (nj - 1).astype(F32)

                @plsc.parallel_loop(0, 16, unroll=1)
                def g_body(g):
                    s0 = g * 16
                    offx = off_v[par, nrow, pl.ds(s0, 16)]
                    offy = off_v[par, nrow + 27, pl.ds(s0, 16)]
                    offz = off_v[par, nrow + 54, pl.ds(s0, 16)]
                    px = offx + (hf + float(j))
                    py = offy + (g.astype(F32) + pyc + 1.0)
                    pz = offz + (iota_f + float(k))
                    q0x, q1x, gx0, gx1 = dim_stuff(px)
                    q0y, q1y, gy0, gy1 = dim_stuff(py)
                    q0z, q1z, gz0, gz1 = dim_stuff(pz)
                    bx0 = q0x * 324
                    bx1 = q1x * 324
                    by0 = q0y * 18
                    by1 = q1y * 18
                    bases = []
                    wts = []
                    for bx, gx in ((bx0, gx0), (bx1, gx1)):
                        for by, gy in ((by0, gy0), (by1, gy1)):
                            bxy = bx + by
                            gxy = gx * gy
                            for bz, gz in ((q0z, gz0), (q1z, gz1)):
                                bases.append(bxy + bz)
                                wts.append(gxy * gz)
                    tidx = 3 * iota_i + (48 * g + nj) + pz_ * 6144
                    for cp in range(4):
                        w0 = plsc.load_gather(tabs[cp], [bases[0]])
                        acc0 = wts[0] * plsc.bitcast(w0 << 16, F32)
                        acc1 = wts[0] * plsc.bitcast(w0, F32)
                        for cor in range(1, 8):
                            w = plsc.load_gather(tabs[cp], [bases[cor]])
                            acc0 = acc0 + wts[cor] * plsc.bitcast(w << 16, F32)
                            acc1 = acc1 + wts[cor] * plsc.bitcast(w, F32)
                        plsc.store_scatter(zb_v, [tidx + (2 * cp) * 768], acc0)
                        plsc.store_scatter(zb_v, [tidx + (2 * cp + 1) * 768],
                                           acc1)
                return 0

            lax.fori_loop(0, 3, nj_body, 0)

            pltpu.make_async_copy(
                zb_v.at[pl.ds(pz_ * 6144, 6144)],
                z_hbm.at[b, h, jk, cg], sem_p).start()
        return 0

    lax.fori_loop(0, 4, hp_body, 0)

    for sem_p in (sem_z, sem_z2):
        pltpu.make_async_copy(zb_v.at[pl.ds(0, 6144)],
                              z_hbm.at[b, h0, 0, cg], sem_p).wait()


def _tc_matmul_body(w_ref, z_ref, o_ref):
    o_ref[0, 0] = jnp.dot(w_ref[...], z_ref[0, 0],
                          preferred_element_type=F32)


@jax.jit
def kernel(x, offset, W):
    xp = jnp.pad(x, ((0, 0), (0, 0), (1, 1), (1, 1), (1, 1)))
    xb = jax.lax.bitcast_convert_type(
        xp.astype(jnp.bfloat16), jnp.uint16).astype(jnp.uint32)
    xb = xb.reshape(2, 16, 2, 5832)
    xt = (xb[:, :, 0] | (xb[:, :, 1] << 16)).astype(jnp.int32).reshape(8, 4, 5832)
    off_t = offset.reshape(2, 81, 16, 256).transpose(0, 2, 1, 3)

    mesh = plsc.VectorSubcoreMesh(core_axis_name="c", subcore_axis_name="s")
    sc = pl.kernel(
        _sc_gather_kernel, mesh=mesh,
        compiler_params=pltpu.CompilerParams(needs_layout_passes=False),
        out_type=jax.ShapeDtypeStruct((2, 16, 9, 4, 6144), F32),
        scratch_types=[
            pltpu.VMEM((5832,), I32),
            pltpu.VMEM((5832,), I32),
            pltpu.VMEM((5832,), I32),
            pltpu.VMEM((5832,), I32),
            pltpu.VMEM((2, 81, 256), F32),
            pltpu.VMEM((12288,), F32),
            pltpu.SemaphoreType.DMA,
            pltpu.SemaphoreType.DMA,
            pltpu.SemaphoreType.DMA,
            pltpu.SemaphoreType.DMA,
        ])
    z = sc(xt, off_t)
    zr = z.reshape(2, 16, 864, 256)

    W3 = W.transpose(0, 3, 4, 1, 2).reshape(64, 864)

    out_t = pl.pallas_call(
        _tc_matmul_body,
        grid=(2, 16),
        in_specs=[
            pl.BlockSpec((64, 864), lambda b, h: (0, 0)),
            pl.BlockSpec((1, 1, 864, 256), lambda b, h: (b, h, 0, 0)),
        ],
        out_specs=pl.BlockSpec((1, 1, 64, 256), lambda b, h: (b, h, 0, 0)),
        out_shape=jax.ShapeDtypeStruct((2, 16, 64, 256), F32),
    )(W3, zr)
    return out_t.transpose(0, 2, 1, 3).reshape(2, 64, 16, 16, 16)


# speedup vs baseline: 1.2367x; 1.0591x over previous
import functools

import jax
import jax.numpy as jnp
from jax import lax
from jax.experimental import pallas as pl
from jax.experimental.pallas import tpu as pltpu
from jax.experimental.pallas import tpu_sc as plsc

F32 = jnp.float32
I32 = jnp.int32


def _sc_gather_kernel(xt_hbm, off_hbm, z_hbm, t0_v, t1_v, t2_v, t3_v,
                      off_v, zb_v, sem_tab, sem_off, sem_z, sem_z2):
    wid = lax.axis_index("s") * 2 + lax.axis_index("c")
    b = wid // 16
    cg = (wid // 4) % 4
    hq = wid % 4
    h0 = hq * 4
    tabs = (t0_v, t1_v, t2_v, t3_v)

    for cp in range(4):
        pltpu.make_async_copy(xt_hbm.at[b * 4 + cg, cp], tabs[cp],
                              sem_tab).start()
    for cp in range(4):
        pltpu.make_async_copy(xt_hbm.at[b * 4 + cg, cp], tabs[cp],
                              sem_tab).wait()

    iota_i = lax.broadcasted_iota(I32, (16,), 0)
    iota_f = iota_i.astype(F32)

    pltpu.make_async_copy(off_hbm.at[b, :, h0, :], off_v.at[0], sem_off).start()

    def dim_stuff(p):
        t = p.astype(I32)
        fl = t - (t.astype(F32) > p).astype(I32)
        flf = fl.astype(F32)
        q0 = jnp.clip(fl, 0, 17)
        q1 = jnp.clip(fl + 1, 0, 17)
        mask = (p < 1.0) | (p > 16.0)
        pm = jnp.where(mask, flf, p)
        pm = jnp.clip(pm, 0.0, 17.0)
        g0 = 1.0 + (q0.astype(F32) - pm)
        g1 = 1.0 - (q1.astype(F32) - pm)
        return q0, q1, g0, g1

    def hp_body(hp, _):
        h = h0 + hp
        par = lax.rem(hp, 2)
        pltpu.make_async_copy(off_hbm.at[b, :, h, :], off_v.at[par], sem_off).wait()

        @pl.when(hp < 3)
        def _():
            pltpu.make_async_copy(off_hbm.at[b, :, h + 1, :],
                                  off_v.at[lax.rem(hp + 1, 2)],
                                  sem_off).start()

        hf = h.astype(F32)

        for jk in range(9):
            j, k = jk // 3, jk % 3
            pz_ = jk % 2
            sem_p = sem_z if pz_ == 0 else sem_z2

            def _reclaim():
                pltpu.make_async_copy(
                    zb_v.at[pl.ds(pz_ * 6144, 6144)],
                    z_hbm.at[b, h, jk, cg], sem_p).wait()

            if jk >= 2:
                _reclaim()
            else:
                pl.when(hp > 0)(_reclaim)

            def nj_body(nj, _):
                nrow = 9 * j + 3 * nj + k
                pyc = (nj - 1).astype(F32)

                @plsc.parallel_loop(0, 16, unroll=1)
                def g_body(g):
                    s0 = g * 16
                    offx = off_v[par, nrow, pl.ds(s0, 16)]
                    offy = off_v[par, nrow + 27, pl.ds(s0, 16)]
                    offz = off_v[par, nrow + 54, pl.ds(s0, 16)]
                    px = offx + (hf + float(j))
                    py = offy + (g.astype(F32) + pyc + 1.0)
                    pz = offz + (iota_f + float(k))
                    q0x, q1x, gx0, gx1 = dim_stuff(px)
                    q0y, q1y, gy0, gy1 = dim_stuff(py)
                    q0z, q1z, gz0, gz1 = dim_stuff(pz)
                    bx0 = q0x * 324
                    bx1 = q1x * 324
                    by0 = q0y * 18
                    by1 = q1y * 18
                    bases = []
                    wts = []
                    for bx, gx in ((bx0, gx0), (bx1, gx1)):
                        for by, gy in ((by0, gy0), (by1, gy1)):
                            bxy = bx + by
                            gxy = gx * gy
                            for bz, gz in ((q0z, gz0), (q1z, gz1)):
                                bases.append(bxy + bz)
                                wts.append(gxy * gz)
                    tidx = 3 * iota_i + (48 * g + nj) + pz_ * 6144
                    for cp in range(4):
                        w0 = plsc.load_gather(tabs[cp], [bases[0]])
                        acc0 = wts[0] * plsc.bitcast(w0 << 16, F32)
                        acc1 = wts[0] * plsc.bitcast(w0, F32)
                        for cor in range(1, 8):
                            w = plsc.load_gather(tabs[cp], [bases[cor]])
                            acc0 = acc0 + wts[cor] * plsc.bitcast(w << 16, F32)
                            acc1 = acc1 + wts[cor] * plsc.bitcast(w, F32)
                        plsc.store_scatter(zb_v, [tidx + (2 * cp) * 768], acc0)
                        plsc.store_scatter(zb_v, [tidx + (2 * cp + 1) * 768],
                                           acc1)
                return 0

            lax.fori_loop(0, 3, nj_body, 0)

            pltpu.make_async_copy(
                zb_v.at[pl.ds(pz_ * 6144, 6144)],
                z_hbm.at[b, h, jk, cg], sem_p).start()
        return 0

    lax.fori_loop(0, 4, hp_body, 0)

    for sem_p in (sem_z, sem_z2):
        pltpu.make_async_copy(zb_v.at[pl.ds(0, 6144)],
                              z_hbm.at[b, h0, 0, cg], sem_p).wait()


def _tc_matmul_body(w_ref, z_ref, o_ref):
    for hh in range(4):
        o_ref[0, hh] = jnp.dot(w_ref[...], z_ref[0, hh],
                               preferred_element_type=F32)


@jax.jit
def kernel(x, offset, W):
    xp = jnp.pad(x, ((0, 0), (0, 0), (1, 1), (1, 1), (1, 1)))
    xb = jax.lax.bitcast_convert_type(
        xp.astype(jnp.bfloat16), jnp.uint16).astype(jnp.uint32)
    xb = xb.reshape(2, 16, 2, 5832)
    xt = (xb[:, :, 0] | (xb[:, :, 1] << 16)).astype(jnp.int32).reshape(8, 4, 5832)
    off_t = offset.reshape(2, 81, 16, 256)

    mesh = plsc.VectorSubcoreMesh(core_axis_name="c", subcore_axis_name="s")
    sc = pl.kernel(
        _sc_gather_kernel, mesh=mesh,
        compiler_params=pltpu.CompilerParams(needs_layout_passes=False),
        out_type=jax.ShapeDtypeStruct((2, 16, 9, 4, 6144), F32),
        scratch_types=[
            pltpu.VMEM((5832,), I32),
            pltpu.VMEM((5832,), I32),
            pltpu.VMEM((5832,), I32),
            pltpu.VMEM((5832,), I32),
            pltpu.VMEM((2, 81, 256), F32),
            pltpu.VMEM((12288,), F32),
            pltpu.SemaphoreType.DMA,
            pltpu.SemaphoreType.DMA,
            pltpu.SemaphoreType.DMA,
            pltpu.SemaphoreType.DMA,
        ])
    z = sc(xt, off_t)
    zr = z.reshape(2, 16, 864, 256)

    W3 = W.transpose(0, 3, 4, 1, 2).reshape(64, 864)

    out_t = pl.pallas_call(
        _tc_matmul_body,
        grid=(2, 4),
        in_specs=[
            pl.BlockSpec((64, 864), lambda b, h: (0, 0)),
            pl.BlockSpec((1, 4, 864, 256), lambda b, h: (b, h, 0, 0)),
        ],
        out_specs=pl.BlockSpec((1, 4, 64, 256), lambda b, h: (b, h, 0, 0)),
        out_shape=jax.ShapeDtypeStruct((2, 16, 64, 256), F32),
    )(W3, zr)
    return out_t.transpose(0, 2, 1, 3).reshape(2, 64, 16, 16, 16)


# speedup vs baseline: 1.2371x; 1.0004x over previous
import functools

import jax
import jax.numpy as jnp
from jax import lax
from jax.experimental import pallas as pl
from jax.experimental.pallas import tpu as pltpu
from jax.experimental.pallas import tpu_sc as plsc

F32 = jnp.float32
I32 = jnp.int32


def _sc_gather_kernel(xt_hbm, off_hbm, z_hbm, t0_v, t1_v, t2_v, t3_v,
                      off_v, zb_v, sem_tab, sem_off, sem_z, sem_z2):
    wid = lax.axis_index("s") * 2 + lax.axis_index("c")
    b = wid // 16
    cg = (wid // 4) % 4
    hq = wid % 4
    h0 = hq * 4
    tabs = (t0_v, t1_v, t2_v, t3_v)

    for cp in range(4):
        pltpu.make_async_copy(xt_hbm.at[b * 4 + cg, cp], tabs[cp],
                              sem_tab).start()
    for cp in range(4):
        pltpu.make_async_copy(xt_hbm.at[b * 4 + cg, cp], tabs[cp],
                              sem_tab).wait()

    iota_i = lax.broadcasted_iota(I32, (16,), 0)
    iota_f = iota_i.astype(F32)

    pltpu.make_async_copy(off_hbm.at[b, :, h0, :], off_v.at[0], sem_off).start()

    def dim_stuff(p):
        t = p.astype(I32)
        fl = t - (t.astype(F32) > p).astype(I32)
        flf = fl.astype(F32)
        q0 = jnp.clip(fl, 0, 17)
        q1 = jnp.clip(fl + 1, 0, 17)
        mask = (p < 1.0) | (p > 16.0)
        pm = jnp.where(mask, flf, p)
        pm = jnp.clip(pm, 0.0, 17.0)
        g0 = 1.0 + (q0.astype(F32) - pm)
        g1 = 1.0 - (q1.astype(F32) - pm)
        return q0, q1, g0, g1

    def hp_body(hp, _):
        h = h0 + hp
        par = lax.rem(hp, 2)
        pltpu.make_async_copy(off_hbm.at[b, :, h, :], off_v.at[par], sem_off).wait()

        @pl.when(hp < 3)
        def _():
            pltpu.make_async_copy(off_hbm.at[b, :, h + 1, :],
                                  off_v.at[lax.rem(hp + 1, 2)],
                                  sem_off).start()

        hf = h.astype(F32)

        for jk in range(9):
            j, k = jk // 3, jk % 3
            pz_ = jk % 2
            sem_p = sem_z if pz_ == 0 else sem_z2

            def _reclaim():
                pltpu.make_async_copy(
                    zb_v.at[pl.ds(pz_ * 6144, 6144)],
                    z_hbm.at[b, h, jk, cg], sem_p).wait()

            if jk >= 2:
                _reclaim()
            else:
                pl.when(hp > 0)(_reclaim)

            def nj_body(nj, _):
                nrow = 9 * j + 3 * nj + k
                pyc = (nj - 1).astype(F32)

                @plsc.parallel_loop(0, 16, unroll=1)
                def g_body(g):
                    s0 = g * 16
                    offx = off_v[par, nrow, pl.ds(s0, 16)]
                    offy = off_v[par, nrow + 27, pl.ds(s0, 16)]
                    offz = off_v[par, nrow + 54, pl.ds(s0, 16)]
                    px = offx + (hf + float(j))
                    py = offy + (g.astype(F32) + pyc + 1.0)
                    pz = offz + (iota_f + float(k))
                    q0x, q1x, gx0, gx1 = dim_stuff(px)
                    q0y, q1y, gy0, gy1 = dim_stuff(py)
                    q0z, q1z, gz0, gz1 = dim_stuff(pz)
                    bx0 = q0x * 324
                    bx1 = q1x * 324
                    by0 = q0y * 18
                    by1 = q1y * 18
                    bases = []
                    wts = []
                    for bx, gx in ((bx0, gx0), (bx1, gx1)):
                        for by, gy in ((by0, gy0), (by1, gy1)):
                            bxy = bx + by
                            gxy = gx * gy
                            for bz, gz in ((q0z, gz0), (q1z, gz1)):
                                bases.append(bxy + bz)
                                wts.append(gxy * gz)
                    tidx = 3 * iota_i + (48 * g + nj) + pz_ * 6144
                    for cp in range(4):
                        w0 = plsc.load_gather(tabs[cp], [bases[0]])
                        acc0 = wts[0] * plsc.bitcast(w0 << 16, F32)
                        acc1 = wts[0] * plsc.bitcast(w0, F32)
                        for cor in range(1, 8):
                            w = plsc.load_gather(tabs[cp], [bases[cor]])
                            acc0 = acc0 + wts[cor] * plsc.bitcast(w << 16, F32)
                            acc1 = acc1 + wts[cor] * plsc.bitcast(w, F32)
                        plsc.store_scatter(zb_v, [tidx + (2 * cp) * 768], acc0)
                        plsc.store_scatter(zb_v, [tidx + (2 * cp + 1) * 768],
                                           acc1)
                return 0

            lax.fori_loop(0, 3, nj_body, 0)

            pltpu.make_async_copy(
                zb_v.at[pl.ds(pz_ * 6144, 6144)],
                z_hbm.at[b, h, jk, cg], sem_p).start()
        return 0

    lax.fori_loop(0, 4, hp_body, 0)

    for sem_p in (sem_z, sem_z2):
        pltpu.make_async_copy(zb_v.at[pl.ds(0, 6144)],
                              z_hbm.at[b, h0, 0, cg], sem_p).wait()


def _tc_matmul_body(w_ref, z_ref, o_ref):
    for hh in range(16):
        o_ref[0, hh] = jnp.dot(w_ref[...], z_ref[0, hh],
                               preferred_element_type=F32)


@jax.jit
def kernel(x, offset, W):
    xp = jnp.pad(x, ((0, 0), (0, 0), (1, 1), (1, 1), (1, 1)))
    xb = jax.lax.bitcast_convert_type(
        xp.astype(jnp.bfloat16), jnp.uint16).astype(jnp.uint32)
    xb = xb.reshape(2, 16, 2, 5832)
    xt = (xb[:, :, 0] | (xb[:, :, 1] << 16)).astype(jnp.int32).reshape(8, 4, 5832)
    off_t = offset.reshape(2, 81, 16, 256)

    mesh = plsc.VectorSubcoreMesh(core_axis_name="c", subcore_axis_name="s")
    sc = pl.kernel(
        _sc_gather_kernel, mesh=mesh,
        compiler_params=pltpu.CompilerParams(needs_layout_passes=False),
        out_type=jax.ShapeDtypeStruct((2, 16, 9, 4, 6144), F32),
        scratch_types=[
            pltpu.VMEM((5832,), I32),
            pltpu.VMEM((5832,), I32),
            pltpu.VMEM((5832,), I32),
            pltpu.VMEM((5832,), I32),
            pltpu.VMEM((2, 81, 256), F32),
            pltpu.VMEM((12288,), F32),
            pltpu.SemaphoreType.DMA,
            pltpu.SemaphoreType.DMA,
            pltpu.SemaphoreType.DMA,
            pltpu.SemaphoreType.DMA,
        ])
    z = sc(xt, off_t)
    zr = z.reshape(2, 16, 864, 256)

    W3 = W.transpose(0, 3, 4, 1, 2).reshape(64, 864)

    out_t = pl.pallas_call(
        _tc_matmul_body,
        grid=(2,),
        in_specs=[
            pl.BlockSpec((64, 864), lambda b: (0, 0)),
            pl.BlockSpec((1, 16, 864, 256), lambda b: (b, 0, 0, 0)),
        ],
        out_specs=pl.BlockSpec((1, 16, 64, 256), lambda b: (b, 0, 0, 0)),
        out_shape=jax.ShapeDtypeStruct((2, 16, 64, 256), F32),
    )(W3, zr)
    return out_t.transpose(0, 2, 1, 3).reshape(2, 64, 16, 16, 16)


# speedup vs baseline: 1.2372x; 1.0000x over previous
import jax
import jax.numpy as jnp
from jax import lax
from jax.experimental import pallas as pl
from jax.experimental.pallas import tpu as pltpu
from jax.experimental.pallas import tpu_sc as plsc

F32 = jnp.float32
I32 = jnp.int32


def _sc_gather_kernel(xt_hbm, off_hbm, z_hbm, t0_v, t1_v, t2_v, t3_v,
                      off_v, zb_v, sem_tab, sem_off, sem_z, sem_z2):
    wid = lax.axis_index("s") * 2 + lax.axis_index("c")
    b = wid // 16
    cg = (wid // 4) % 4
    hq = wid % 4
    h0 = hq * 4
    tabs = (t0_v, t1_v, t2_v, t3_v)

    for cp in range(4):
        pltpu.make_async_copy(xt_hbm.at[b * 4 + cg, cp], tabs[cp],
                              sem_tab).start()
    for cp in range(4):
        pltpu.make_async_copy(xt_hbm.at[b * 4 + cg, cp], tabs[cp],
                              sem_tab).wait()

    iota_i = lax.broadcasted_iota(I32, (16,), 0)
    iota_f = iota_i.astype(F32)

    pltpu.make_async_copy(off_hbm.at[b, :, h0, :], off_v.at[0], sem_off).start()

    def dim_stuff(p):
        t = p.astype(I32)
        fl = t - (t.astype(F32) > p).astype(I32)
        flf = fl.astype(F32)
        q0 = jnp.clip(fl, 0, 17)
        q1 = jnp.clip(fl + 1, 0, 17)
        mask = (p < 1.0) | (p > 16.0)
        pm = jnp.where(mask, flf, p)
        pm = jnp.clip(pm, 0.0, 17.0)
        g0 = 1.0 + (q0.astype(F32) - pm)
        g1 = 1.0 - (q1.astype(F32) - pm)
        return q0, q1, g0, g1

    def hp_body(hp, _):
        h = h0 + hp
        par = lax.rem(hp, 2)
        pltpu.make_async_copy(off_hbm.at[b, :, h, :], off_v.at[par], sem_off).wait()

        @pl.when(hp < 3)
        def _():
            pltpu.make_async_copy(off_hbm.at[b, :, h + 1, :],
                                  off_v.at[lax.rem(hp + 1, 2)],
                                  sem_off).start()

        hf = h.astype(F32)

        for jk in range(9):
            j, k = jk // 3, jk % 3
            pz_ = jk % 2
            sem_p = sem_z if pz_ == 0 else sem_z2

            def _reclaim():
                pltpu.make_async_copy(
                    zb_v.at[pl.ds(pz_ * 6144, 6144)],
                    z_hbm.at[b, h, jk, cg], sem_p).wait()

            if jk >= 2:
                _reclaim()
            else:
                pl.when(hp > 0)(_reclaim)

            def nj_body(nj, _):
                nrow = 9 * j + 3 * nj + k
                pyc = (nj - 1).astype(F32)

                @plsc.parallel_loop(0, 16, unroll=1)
                def g_body(g):
                    s0 = g * 16
                    offx = off_v[par, nrow, pl.ds(s0, 16)]
                    offy = off_v[par, nrow + 27, pl.ds(s0, 16)]
                    offz = off_v[par, nrow + 54, pl.ds(s0, 16)]
                    px = offx + (hf + float(j))
                    py = offy + (g.astype(F32) + pyc + 1.0)
                    pz = offz + (iota_f + float(k))
                    q0x, q1x, gx0, gx1 = dim_stuff(px)
                    q0y, q1y, gy0, gy1 = dim_stuff(py)
                    q0z, q1z, gz0, gz1 = dim_stuff(pz)
                    bx0 = q0x * 324
                    bx1 = q1x * 324
                    by0 = q0y * 18
                    by1 = q1y * 18
                    bases = []
                    wts = []
                    for bx, gx in ((bx0, gx0), (bx1, gx1)):
                        for by, gy in ((by0, gy0), (by1, gy1)):
                            bxy = bx + by
                            gxy = gx * gy
                            for bz, gz in ((q0z, gz0), (q1z, gz1)):
                                bases.append(bxy + bz)
                                wts.append(gxy * gz)
                    tidx = 3 * iota_i + (48 * g + nj) + pz_ * 6144
                    for cp in range(4):
                        w0 = plsc.load_gather(tabs[cp], [bases[0]])
                        acc0 = wts[0] * plsc.bitcast(w0 << 16, F32)
                        acc1 = wts[0] * plsc.bitcast(w0, F32)
                        for cor in range(1, 8):
                            w = plsc.load_gather(tabs[cp], [bases[cor]])
                            acc0 = acc0 + wts[cor] * plsc.bitcast(w << 16, F32)
                            acc1 = acc1 + wts[cor] * plsc.bitcast(w, F32)
                        plsc.store_scatter(zb_v, [tidx + (2 * cp) * 768], acc0)
                        plsc.store_scatter(zb_v, [tidx + (2 * cp + 1) * 768],
                                           acc1)
                return 0

            lax.fori_loop(0, 3, nj_body, 0)

            pltpu.make_async_copy(
                zb_v.at[pl.ds(pz_ * 6144, 6144)],
                z_hbm.at[b, h, jk, cg], sem_p).start()
        return 0

    lax.fori_loop(0, 4, hp_body, 0)

    for sem_p in (sem_z, sem_z2):
        pltpu.make_async_copy(zb_v.at[pl.ds(0, 6144)],
                              z_hbm.at[b, h0, 0, cg], sem_p).wait()


def _tc_matmul_body(w_ref, z_ref, o_ref):
    for hh in range(16):
        o_ref[0, hh] = jnp.dot(w_ref[...], z_ref[0, hh],
                               preferred_element_type=F32)


@jax.jit
def kernel(x, offset, W):
    xp = jnp.pad(x, ((0, 0), (0, 0), (1, 1), (1, 1), (1, 1)))
    xb = jax.lax.bitcast_convert_type(
        xp.astype(jnp.bfloat16), jnp.uint16).astype(jnp.uint32)
    xb = xb.reshape(2, 16, 2, 5832)
    xt = (xb[:, :, 0] | (xb[:, :, 1] << 16)).astype(jnp.int32).reshape(8, 4, 5832)
    off_t = offset.reshape(2, 81, 16, 256)

    mesh = plsc.VectorSubcoreMesh(core_axis_name="c", subcore_axis_name="s")
    sc = pl.kernel(
        _sc_gather_kernel, mesh=mesh,
        compiler_params=pltpu.CompilerParams(needs_layout_passes=False),
        out_type=jax.ShapeDtypeStruct((2, 16, 9, 4, 6144), F32),
        scratch_types=[
            pltpu.VMEM((5832,), I32),
            pltpu.VMEM((5832,), I32),
            pltpu.VMEM((5832,), I32),
            pltpu.VMEM((5832,), I32),
            pltpu.VMEM((2, 81, 256), F32),
            pltpu.VMEM((12288,), F32),
            pltpu.SemaphoreType.DMA,
            pltpu.SemaphoreType.DMA,
            pltpu.SemaphoreType.DMA,
            pltpu.SemaphoreType.DMA,
        ])
    z = sc(xt, off_t)
    zr = z.reshape(2, 16, 864, 256)

    W3 = W.transpose(0, 3, 4, 1, 2).reshape(64, 864)

    out_t = pl.pallas_call(
        _tc_matmul_body,
        grid=(2,),
        in_specs=[
            pl.BlockSpec((64, 864), lambda b: (0, 0)),
            pl.BlockSpec((1, 16, 864, 256), lambda b: (b, 0, 0, 0)),
        ],
        out_specs=pl.BlockSpec((1, 16, 64, 256), lambda b: (b, 0, 0, 0)),
        out_shape=jax.ShapeDtypeStruct((2, 16, 64, 256), F32),
    )(W3, zr)
    return out_t.transpose(0, 2, 1, 3).reshape(2, 64, 16, 16, 16)
